# baseline (speedup 1.0000x reference)
import functools

import jax
import jax.numpy as jnp
from jax.experimental import pallas as pl
from jax.experimental.pallas import tpu as pltpu

_K = 16


def _topk_body(sim_ref, mask_ref, idx_ref, val_ref, *, tm):
    nchunks = max(sim_ref.shape[1] // 128, 1)
    x = sim_ref[...]
    ui = pltpu.bitcast(x, jnp.uint32)
    col = jax.lax.broadcasted_iota(jnp.uint32, x.shape, 1)
    chunk_rev = jnp.uint32(nchunks - 1) - (col >> 7)
    enc = (ui & jnp.uint32(0xFFFFFFE0)) | chunk_rev
    y0 = pltpu.bitcast(enc, jnp.float32)
    vcols, icols = [], []
    for p in range(_K):
        y = y0 if p == 0 else sim_ref[...]
        tree = y[:, :128]
        for c in range(1, nchunks):
            tree = jnp.maximum(tree, y[:, c * 128:(c + 1) * 128])
        m = jnp.max(tree, axis=1, keepdims=True)
        lane = jnp.argmax(tree, axis=1).astype(jnp.int32)[:, None]
        sim_ref[...] = jnp.where(y == m, -1.0, y)
        mui = pltpu.bitcast(m, jnp.uint32)
        chunk = jnp.int32(nchunks - 1) - (mui & jnp.uint32(31)).astype(jnp.int32)
        icols.append(chunk * 128 + lane)
        vcols.append(pltpu.bitcast(mui & jnp.uint32(0xFFFFFFE0), jnp.float32))
    vals = jnp.concatenate(vcols, axis=1)
    idx = jnp.concatenate(icols, axis=1)
    denom = jnp.sum(vals, axis=1, keepdims=True)
    mask = mask_ref[...] > 0.0
    scaled = jnp.where(mask, 0.0, vals / denom)
    kcol = jax.lax.broadcasted_iota(jnp.int32, vals.shape, 1)
    scaled = jnp.where(mask & (kcol == 0), 1.0, scaled)
    base = pl.program_id(0) * tm
    rows = base + jax.lax.broadcasted_iota(jnp.int32, idx.shape, 0)
    val_ref[...] = scaled
    idx_ref[...] = jnp.where(mask, rows, idx)


def _topk(sim, mask_f, *, tm=512):
    n, w = sim.shape
    tm = min(tm, n)
    grid = (n // tm,)
    return pl.pallas_call(
        functools.partial(_topk_body, tm=tm),
        out_shape=(jax.ShapeDtypeStruct((n, _K), jnp.int32),
                   jax.ShapeDtypeStruct((n, _K), jnp.float32)),
        grid=grid,
        in_specs=[
            pl.BlockSpec((tm, w), lambda i: (i, 0)),
            pl.BlockSpec((tm, 1), lambda i: (i, 0)),
        ],
        out_specs=(pl.BlockSpec((tm, _K), lambda i: (i, 0)),
                   pl.BlockSpec((tm, _K), lambda i: (i, 0))),
        compiler_params=pltpu.CompilerParams(
            dimension_semantics=("parallel",),
            vmem_limit_bytes=48 * 1024 * 1024),
    )(sim, mask_f)


def _smooth_body(idx_ref, w_ref, feat_ref, out_ref, *scratch, tm, ngrp):
    def chunk(it, carry):
        for half in range(ngrp):
            g_scr = scratch[half]
            base = pl.multiple_of((it * ngrp + half) * 8, 8)
            for rr in range(8):
                for j in range(_K):
                    slot = rr * _K + j
                    g_scr[slot:slot + 1, :] = feat_ref[idx_ref[base + rr, j]]
            wg = w_ref[pl.ds(base, 8), :]
            res = jax.lax.dot_general(wg, g_scr[...],
                                      dimension_numbers=(((1,), (0,)), ((), ())),
                                      preferred_element_type=jnp.float32)
            out_ref[pl.ds(base, 8), :] = res
        return carry

    jax.lax.fori_loop(0, tm // (8 * ngrp), chunk, 0)


def _smooth(idx, w_blockdiag, feat3, *, tm=256, ngrp=4):
    n, _, d = feat3.shape
    tm = min(tm, n)
    while tm % (8 * ngrp):
        ngrp //= 2
    grid = (n // tm,)
    return pl.pallas_call(
        functools.partial(_smooth_body, tm=tm, ngrp=ngrp),
        out_shape=jax.ShapeDtypeStruct((n, d), jnp.float32),
        grid=grid,
        in_specs=[
            pl.BlockSpec((tm, _K), lambda i: (i, 0), memory_space=pltpu.SMEM),
            pl.BlockSpec((tm, 8 * _K), lambda i: (i, 0)),
            pl.BlockSpec((n, 1, d), lambda i: (0, 0, 0)),
        ],
        out_specs=pl.BlockSpec((tm, d), lambda i: (i, 0)),
        scratch_shapes=[pltpu.VMEM((8 * _K, d), jnp.float32)
                        for _ in range(ngrp)],
        compiler_params=pltpu.CompilerParams(
            dimension_semantics=("parallel",),
            vmem_limit_bytes=48 * 1024 * 1024),
    )(idx, w_blockdiag, feat3)


def _decode_body(u_ref, v_ref, out_ref):
    x = jax.lax.dot_general(u_ref[...], v_ref[...],
                            dimension_numbers=(((1,), (1,)), ((), ())),
                            preferred_element_type=jnp.float32)
    out_ref[...] = 0.5 + 0.5 * jnp.tanh(0.5 * x)


def _decode(u, v, *, tm=256, tn=512):
    su, d = u.shape
    sv, _ = v.shape
    tm = min(tm, su)
    tn = min(tn, sv)
    grid = (su // tm, sv // tn)
    return pl.pallas_call(
        _decode_body,
        out_shape=jax.ShapeDtypeStruct((su, sv), jnp.float32),
        grid=grid,
        in_specs=[
            pl.BlockSpec((tm, d), lambda i, j: (i, 0)),
            pl.BlockSpec((tn, d), lambda i, j: (j, 0)),
        ],
        out_specs=pl.BlockSpec((tm, tn), lambda i, j: (i, j)),
        compiler_params=pltpu.CompilerParams(
            dimension_semantics=("parallel", "parallel"),
            vmem_limit_bytes=48 * 1024 * 1024),
    )(u, v)


def kernel(u, v, sim, mask_bool):
    size_u, d = u.shape
    feature = jnp.concatenate([u, v], axis=0).astype(jnp.float32)
    n = feature.shape[0]

    mask_f = mask_bool.reshape(n, 1).astype(jnp.float32)
    idx, scaled = _topk(sim, mask_f)

    lane = jax.lax.broadcasted_iota(jnp.int32, (n, 8 * _K), 1)
    rowmod = (jax.lax.broadcasted_iota(jnp.int32, (n, 1), 0) & 7)
    w_blockdiag = jnp.where((lane >> 4) == rowmod, jnp.tile(scaled, (1, 8)), 0.0)

    smoothed = _smooth(idx, w_blockdiag, feature.reshape(n, 1, d))
    u_new = smoothed[:size_u]
    v_new = smoothed[size_u:]

    outputs = _decode(u_new.astype(jnp.bfloat16), v_new.astype(jnp.bfloat16))
    return outputs, u_new, v_new


# speedup vs baseline: 1.0180x; 1.0180x over previous
import functools

import jax
import jax.numpy as jnp
from jax.experimental import pallas as pl
from jax.experimental.pallas import tpu as pltpu

_K = 16


def _topk_body(sim_ref, mask_ref, idx_ref, val_ref, *, tm):
    nchunks = max(sim_ref.shape[1] // 128, 1)
    x = sim_ref[...]
    ui = pltpu.bitcast(x, jnp.uint32)
    col = jax.lax.broadcasted_iota(jnp.uint32, x.shape, 1)
    chunk_rev = jnp.uint32(nchunks - 1) - (col >> 7)
    enc = (ui & jnp.uint32(0xFFFFFFE0)) | chunk_rev
    y0 = pltpu.bitcast(enc, jnp.float32)
    vcols, icols = [], []
    for p in range(_K):
        y = y0 if p == 0 else sim_ref[...]
        tree = y[:, :128]
        for c in range(1, nchunks):
            tree = jnp.maximum(tree, y[:, c * 128:(c + 1) * 128])
        m = jnp.max(tree, axis=1, keepdims=True)
        lane = jnp.argmax(tree, axis=1).astype(jnp.int32)[:, None]
        sim_ref[...] = jnp.where(y == m, -1.0, y)
        mui = pltpu.bitcast(m, jnp.uint32)
        chunk = jnp.int32(nchunks - 1) - (mui & jnp.uint32(31)).astype(jnp.int32)
        icols.append(chunk * 128 + lane)
        vcols.append(pltpu.bitcast(mui & jnp.uint32(0xFFFFFFE0), jnp.float32))
    vals = jnp.concatenate(vcols, axis=1)
    idx = jnp.concatenate(icols, axis=1)
    denom = jnp.sum(vals, axis=1, keepdims=True)
    mask = mask_ref[...] > 0.0
    scaled = jnp.where(mask, 0.0, vals / denom)
    kcol = jax.lax.broadcasted_iota(jnp.int32, vals.shape, 1)
    scaled = jnp.where(mask & (kcol == 0), 1.0, scaled)
    base = pl.program_id(0) * tm
    rows = base + jax.lax.broadcasted_iota(jnp.int32, idx.shape, 0)
    val_ref[...] = scaled
    idx_ref[...] = jnp.where(mask, rows, idx)


def _topk(sim, mask_f, *, tm=512):
    n, w = sim.shape
    tm = min(tm, n)
    grid = (n // tm,)
    return pl.pallas_call(
        functools.partial(_topk_body, tm=tm),
        out_shape=(jax.ShapeDtypeStruct((n, _K), jnp.int32),
                   jax.ShapeDtypeStruct((n, _K), jnp.float32)),
        grid=grid,
        in_specs=[
            pl.BlockSpec((tm, w), lambda i: (i, 0)),
            pl.BlockSpec((tm, 1), lambda i: (i, 0)),
        ],
        out_specs=(pl.BlockSpec((tm, _K), lambda i: (i, 0)),
                   pl.BlockSpec((tm, _K), lambda i: (i, 0))),
        compiler_params=pltpu.CompilerParams(
            dimension_semantics=("parallel",),
            vmem_limit_bytes=48 * 1024 * 1024),
    )(sim, mask_f)


def _smooth_body(idx_ref, val_ref, feat_ref, out_ref, *, tm, unroll=8):
    def chunk(it, carry):
        r0 = it * unroll
        accs = []
        for uu in range(unroll):
            r = r0 + uu
            acc = val_ref[r, 0] * feat_ref[idx_ref[r, 0]]
            for j in range(1, _K):
                acc = acc + val_ref[r, j] * feat_ref[idx_ref[r, j]]
            accs.append(acc)
        for uu in range(unroll):
            out_ref[r0 + uu] = accs[uu]
        return carry

    jax.lax.fori_loop(0, tm // unroll, chunk, 0)


def _smooth(idx, values, feat3, *, tm=256):
    n, _, d = feat3.shape
    tm = min(tm, n)
    grid = (n // tm,)
    return pl.pallas_call(
        functools.partial(_smooth_body, tm=tm),
        out_shape=jax.ShapeDtypeStruct((n, 1, d), jnp.float32),
        grid=grid,
        in_specs=[
            pl.BlockSpec((tm, _K), lambda i: (i, 0), memory_space=pltpu.SMEM),
            pl.BlockSpec((tm, _K), lambda i: (i, 0), memory_space=pltpu.SMEM),
            pl.BlockSpec((n, 1, d), lambda i: (0, 0, 0)),
        ],
        out_specs=pl.BlockSpec((tm, 1, d), lambda i: (i, 0, 0)),
        compiler_params=pltpu.CompilerParams(
            dimension_semantics=("parallel",),
            vmem_limit_bytes=48 * 1024 * 1024),
    )(idx, values, feat3)


def _decode_body(u_ref, v_ref, out_ref):
    x = jax.lax.dot_general(u_ref[...], v_ref[...],
                            dimension_numbers=(((1,), (1,)), ((), ())),
                            preferred_element_type=jnp.float32)
    out_ref[...] = 0.5 + 0.5 * jnp.tanh(0.5 * x)


def _decode(u, v, *, tm=256, tn=512):
    su, d = u.shape
    sv, _ = v.shape
    tm = min(tm, su)
    tn = min(tn, sv)
    grid = (su // tm, sv // tn)
    return pl.pallas_call(
        _decode_body,
        out_shape=jax.ShapeDtypeStruct((su, sv), jnp.float32),
        grid=grid,
        in_specs=[
            pl.BlockSpec((tm, d), lambda i, j: (i, 0)),
            pl.BlockSpec((tn, d), lambda i, j: (j, 0)),
        ],
        out_specs=pl.BlockSpec((tm, tn), lambda i, j: (i, j)),
        compiler_params=pltpu.CompilerParams(
            dimension_semantics=("parallel", "parallel"),
            vmem_limit_bytes=48 * 1024 * 1024),
    )(u, v)


def kernel(u, v, sim, mask_bool):
    size_u, d = u.shape
    feature = jnp.concatenate([u, v], axis=0).astype(jnp.float32)
    n = feature.shape[0]

    mask_f = mask_bool.reshape(n, 1).astype(jnp.float32)
    idx, scaled = _topk(sim, mask_f)

    out3 = _smooth(idx, scaled, feature.reshape(n, 1, d))
    smoothed = out3.reshape(n, d)
    u_new = smoothed[:size_u]
    v_new = smoothed[size_u:]

    outputs = _decode(u_new.astype(jnp.bfloat16), v_new.astype(jnp.bfloat16))
    return outputs, u_new, v_new
